# Initial kernel scaffold; baseline (speedup 1.0000x reference)
import jax
import jax.numpy as jnp
from jax.experimental import pallas as pl


def kernel(belief, state, Wb, Ws, Wa, Wz, wrb, wrs):
    raise NotImplementedError("write your pallas kernel here")


# speedup vs baseline: 4.8001x; 4.8001x over previous
import jax
import jax.numpy as jnp
from jax.experimental import pallas as pl
from jax.experimental.pallas import tpu as pltpu

_ACTION = 6
_HORIZON = 12
_ITERS = 2
_CAND = 1000
_TOP = 100
_HA = _HORIZON * _ACTION
_NCHUNK = 8
_CHUNK = 128


def _planner_body(belief_ref, state_ref, Wb_ref, Ws_ref, Wa_ref, Wz_ref,
                  wrb_ref, wrs_ref, nh0_ref, nh1_ref, nc0_ref, nc1_ref,
                  out_ref,
                  mean_s, std_s, b_s, s_s, ret_s, rrow_s):
    H = Wb_ref.shape[0]
    Z = Ws_ref.shape[0]
    Wb = Wb_ref[...]
    Ws = Ws_ref[...]
    Wa = Wa_ref[...]
    Wz = Wz_ref[...]
    wrb = wrb_ref[...]
    wrs = wrs_ref[...]

    mean_s[...] = jnp.zeros((_HORIZON, _ACTION), dtype=jnp.float32)
    std_s[...] = jnp.ones((_HORIZON, _ACTION), dtype=jnp.float32)

    hp = jax.lax.Precision.HIGHEST

    for it in range(_ITERS):
        nh_ref = nh0_ref if it == 0 else nh1_ref
        nc_ref = nc0_ref if it == 0 else nc1_ref

        b_s[...] = jnp.broadcast_to(belief_ref[0], (_CAND, H))
        s_s[...] = jnp.broadcast_to(state_ref[0], (_CAND, Z))
        ret_s[...] = jnp.zeros((_CAND, 1), dtype=jnp.float32)

        def roll_step(h, _):
            a = mean_s[h] + std_s[h] * nh_ref[0, h]
            pre = b_s[...] @ Wb + s_s[...] @ Ws + a @ Wa
            bc = jnp.tanh(pre)
            sc = jnp.tanh(bc @ Wz)
            b_s[...] = bc
            s_s[...] = sc
            ret_s[...] += bc @ wrb + sc @ wrs
            return 0

        jax.lax.fori_loop(0, _HORIZON, roll_step, 0)

        ret = ret_s[...]
        rrow_s[...] = jnp.full((1, _NCHUNK * _CHUNK), -jnp.inf,
                               dtype=jnp.float32)
        rrow_s[:, 0:_CAND] = ret.reshape(1, _CAND)

        ii = jax.lax.broadcasted_iota(jnp.int32, (_CAND, _CHUNK), 0)
        jj = jax.lax.broadcasted_iota(jnp.int32, (_CAND, _CHUNK), 1)

        def rank_step(c, rank):
            base = c * _CHUNK
            rj = rrow_s[:, pl.ds(base, _CHUNK)]
            beats = (rj > ret) | ((rj == ret) & (jj + base < ii))
            return rank + jnp.sum(beats.astype(jnp.float32), axis=1,
                                  keepdims=True)

        rank = jax.lax.fori_loop(
            0, _NCHUNK, rank_step,
            jnp.zeros((_CAND, 1), dtype=jnp.float32))
        mask = (rank < float(_TOP)).astype(jnp.float32)
        maskT = mask.reshape(1, _CAND)

        noise_cm = nc_ref[0]
        ssum = jax.lax.dot(maskT, noise_cm, precision=hp)
        mean_n = ssum * (1.0 / _TOP)
        if it + 1 < _ITERS:
            cent = noise_cm - mean_n
            msk2 = jax.lax.dot(maskT, cent * cent, precision=hp)
            std_n = jnp.sqrt(msk2 * (1.0 / _TOP))
            for h in range(_HORIZON):
                c0 = _ACTION * h
                mean_s[h] = mean_s[h] + std_s[h] * mean_n[0, c0:c0 + _ACTION]
                std_s[h] = std_s[h] * std_n[0, c0:c0 + _ACTION]
        else:
            out_ref[0, 0] = mean_s[0] + std_s[0] * mean_n[0, 0:_ACTION]


def kernel(belief, state, Wb, Ws, Wa, Wz, wrb, wrs):
    B, H = belief.shape
    Z = state.shape[1]

    key = jax.random.key(1234)
    nh, nc = [], []
    for i in range(_ITERS):
        sub = jax.random.fold_in(key, i)
        n = jax.random.normal(sub, (_HORIZON, B, _CAND, _ACTION),
                              dtype=jnp.float32)
        nh.append(jnp.transpose(n, (1, 0, 2, 3)))
        nc.append(jnp.transpose(n, (1, 2, 0, 3)).reshape(B, _CAND, _HA))

    full = lambda *shape: pl.BlockSpec(shape, lambda b: (0,) * len(shape))

    out = pl.pallas_call(
        _planner_body,
        grid=(B,),
        in_specs=[
            pl.BlockSpec((1, 1, H), lambda b: (b, 0, 0)),
            pl.BlockSpec((1, 1, Z), lambda b: (b, 0, 0)),
            full(H, H),
            full(Z, H),
            full(_ACTION, H),
            full(H, Z),
            full(H, 1),
            full(Z, 1),
            pl.BlockSpec((1, _HORIZON, _CAND, _ACTION),
                         lambda b: (b, 0, 0, 0)),
            pl.BlockSpec((1, _HORIZON, _CAND, _ACTION),
                         lambda b: (b, 0, 0, 0)),
            pl.BlockSpec((1, _CAND, _HA), lambda b: (b, 0, 0)),
            pl.BlockSpec((1, _CAND, _HA), lambda b: (b, 0, 0)),
        ],
        out_specs=pl.BlockSpec((1, 1, _ACTION), lambda b: (b, 0, 0)),
        out_shape=jax.ShapeDtypeStruct((B, 1, _ACTION), jnp.float32),
        scratch_shapes=[
            pltpu.VMEM((_HORIZON, _ACTION), jnp.float32),
            pltpu.VMEM((_HORIZON, _ACTION), jnp.float32),
            pltpu.VMEM((_CAND, H), jnp.float32),
            pltpu.VMEM((_CAND, Z), jnp.float32),
            pltpu.VMEM((_CAND, 1), jnp.float32),
            pltpu.VMEM((1, _NCHUNK * _CHUNK), jnp.float32),
        ],
    )(belief.reshape(B, 1, H), state.reshape(B, 1, Z), Wb, Ws, Wa, Wz,
      wrb.reshape(H, 1), wrs.reshape(Z, 1), nh[0], nh[1], nc[0], nc[1])
    return out.reshape(B, _ACTION)


# speedup vs baseline: 5.0068x; 1.0431x over previous
import jax
import jax.numpy as jnp
from jax.experimental import pallas as pl
from jax.experimental.pallas import tpu as pltpu

_ACTION = 6
_HORIZON = 12
_ITERS = 2
_CAND = 1000
_TOP = 100
_HA = _HORIZON * _ACTION
_NCHUNK = 8
_CHUNK = 128


def _planner_body(belief_ref, state_ref, Wb_ref, Ws_ref, Wa_ref, Wz_ref,
                  wrb_ref, wrs_ref, nh0_ref, nh1_ref, nc0_ref,
                  out_ref,
                  mean_s, std_s, b_s, s_s, ret_s, rrow_s):
    H = Wb_ref.shape[0]
    Wb = Wb_ref[...]
    Ws = Ws_ref[...]
    Wa = Wa_ref[...]
    Wz = Wz_ref[...]
    wrb = wrb_ref[...]
    wrs = wrs_ref[...]

    hp = jax.lax.Precision.HIGHEST

    pre_shared = belief_ref[0] @ Wb + state_ref[0] @ Ws

    ii = jax.lax.broadcasted_iota(jnp.int32, (_CAND, _CHUNK), 0)
    jj = jax.lax.broadcasted_iota(jnp.int32, (_CAND, _CHUNK), 1)

    for it in range(_ITERS):
        nh_ref = nh0_ref if it == 0 else nh1_ref

        if it == 0:
            a0 = nh_ref[0, 0]
        else:
            a0 = mean_s[0] + std_s[0] * nh_ref[0, 0]
        bc = jnp.tanh(pre_shared + a0 @ Wa)
        sc = jnp.tanh(bc @ Wz)
        b_s[...] = bc
        s_s[...] = sc
        ret_s[...] = bc @ wrb + sc @ wrs

        def roll_step(h, _):
            if it == 0:
                a = nh_ref[0, h]
            else:
                a = mean_s[h] + std_s[h] * nh_ref[0, h]
            pre = b_s[...] @ Wb + s_s[...] @ Ws + a @ Wa
            bc = jnp.tanh(pre)
            sc = jnp.tanh(bc @ Wz)
            b_s[...] = bc
            s_s[...] = sc
            ret_s[...] += bc @ wrb + sc @ wrs
            return 0

        jax.lax.fori_loop(1, _HORIZON, roll_step, 0)

        ret = ret_s[...]

        rrow_s[...] = jnp.full((1, _NCHUNK * _CHUNK), -jnp.inf,
                               dtype=jnp.float32)
        rrow_s[:, 0:_CAND] = ret.reshape(1, _CAND)

        def rank_step(c, rank):
            base = c * _CHUNK
            rj = rrow_s[:, pl.ds(base, _CHUNK)]
            beats = (rj > ret) | ((rj == ret) & (jj + base < ii))
            return rank + jnp.sum(beats.astype(jnp.float32), axis=1,
                                  keepdims=True)

        rank = jax.lax.fori_loop(
            0, _NCHUNK, rank_step,
            jnp.zeros((_CAND, 1), dtype=jnp.float32))
        maskT = (rank < float(_TOP)).astype(jnp.float32).reshape(1, _CAND)

        if it + 1 < _ITERS:
            noise_cm = nc0_ref[0]
            ssum = jax.lax.dot(maskT, noise_cm, precision=hp)
            mean_n = ssum * (1.0 / _TOP)
            cent = noise_cm - mean_n
            msk2 = jax.lax.dot(maskT, cent * cent, precision=hp)
            std_n = jnp.sqrt(msk2 * (1.0 / _TOP))
            for h in range(_HORIZON):
                c0 = _ACTION * h
                mean_s[h] = mean_n[0, c0:c0 + _ACTION]
                std_s[h] = std_n[0, c0:c0 + _ACTION]
        else:
            s6 = jax.lax.dot(maskT, nh_ref[0, 0], precision=hp)
            out_ref[0, 0] = mean_s[0] + std_s[0] * (s6[0] * (1.0 / _TOP))


def kernel(belief, state, Wb, Ws, Wa, Wz, wrb, wrs):
    B, H = belief.shape
    Z = state.shape[1]

    key = jax.random.key(1234)
    nh = []
    for i in range(_ITERS):
        sub = jax.random.fold_in(key, i)
        n = jax.random.normal(sub, (_HORIZON, B, _CAND, _ACTION),
                              dtype=jnp.float32)
        nh.append(jnp.transpose(n, (1, 0, 2, 3)))
    nc0 = jnp.transpose(nh[0], (0, 2, 1, 3)).reshape(B, _CAND, _HA)

    full = lambda *shape: pl.BlockSpec(shape, lambda b: (0,) * len(shape))

    out = pl.pallas_call(
        _planner_body,
        grid=(B,),
        in_specs=[
            pl.BlockSpec((1, 1, H), lambda b: (b, 0, 0)),
            pl.BlockSpec((1, 1, Z), lambda b: (b, 0, 0)),
            full(H, H),
            full(Z, H),
            full(_ACTION, H),
            full(H, Z),
            full(H, 1),
            full(Z, 1),
            pl.BlockSpec((1, _HORIZON, _CAND, _ACTION),
                         lambda b: (b, 0, 0, 0)),
            pl.BlockSpec((1, _HORIZON, _CAND, _ACTION),
                         lambda b: (b, 0, 0, 0)),
            pl.BlockSpec((1, _CAND, _HA), lambda b: (b, 0, 0)),
        ],
        out_specs=pl.BlockSpec((1, 1, _ACTION), lambda b: (b, 0, 0)),
        out_shape=jax.ShapeDtypeStruct((B, 1, _ACTION), jnp.float32),
        scratch_shapes=[
            pltpu.VMEM((_HORIZON, _ACTION), jnp.float32),
            pltpu.VMEM((_HORIZON, _ACTION), jnp.float32),
            pltpu.VMEM((_CAND, H), jnp.float32),
            pltpu.VMEM((_CAND, Z), jnp.float32),
            pltpu.VMEM((_CAND, 1), jnp.float32),
            pltpu.VMEM((1, _NCHUNK * _CHUNK), jnp.float32),
        ],
    )(belief.reshape(B, 1, H), state.reshape(B, 1, Z), Wb, Ws, Wa, Wz,
      wrb.reshape(H, 1), wrs.reshape(Z, 1), nh[0], nh[1], nc0)
    return out.reshape(B, _ACTION)


# speedup vs baseline: 6.2984x; 1.2580x over previous
import jax
import jax.numpy as jnp
from jax.experimental import pallas as pl
from jax.experimental.pallas import tpu as pltpu

_ACTION = 6
_HORIZON = 12
_ITERS = 2
_CAND = 1000
_TOP = 100
_HA = _HORIZON * _ACTION
_NCHUNK = 8
_CHUNK = 128
_H = 200
_Z = 30


def _planner_body(belief_ref, state_ref, WbZ_ref, WsE_ref, Wa_ref, wrs_ref,
                  nh0_ref, nh1_ref, nc0_ref,
                  out_ref,
                  mean_s, std_s, rrow_s):
    WbZ = WbZ_ref[...]
    WsE = WsE_ref[...]
    Wa = Wa_ref[...]
    wrs = wrs_ref[...]

    hp = jax.lax.Precision.HIGHEST

    pre_shared = (belief_ref[0] @ WbZ[:, 0:_H]
                  + state_ref[0] @ WsE[:, 0:_H])

    ii = jax.lax.broadcasted_iota(jnp.int32, (_CAND, _CHUNK), 0)
    jj = jax.lax.broadcasted_iota(jnp.int32, (_CAND, _CHUNK), 1)

    for it in range(_ITERS):
        nh_ref = nh0_ref if it == 0 else nh1_ref

        if it == 0:
            a0 = nh_ref[0, 0]
        else:
            a0 = mean_s[0] + std_s[0] * nh_ref[0, 0]
        bc = jnp.tanh(pre_shared + a0 @ Wa)
        zz = bc @ WbZ
        sc = jnp.tanh(zz[:, _H:_H + _Z])
        bW = zz[:, 0:_H]
        rb = zz[:, _H + _Z:_H + _Z + 1]
        ret = None

        for h in range(1, _HORIZON):
            ss = sc @ WsE
            step_r = rb + ss[:, _H:_H + 1]
            ret = step_r if ret is None else ret + step_r
            if it == 0:
                a = nh_ref[0, h]
            else:
                a = mean_s[h] + std_s[h] * nh_ref[0, h]
            pre = bW + ss[:, 0:_H] + a @ Wa
            bc = jnp.tanh(pre)
            zz = bc @ WbZ
            sc = jnp.tanh(zz[:, _H:_H + _Z])
            bW = zz[:, 0:_H]
            rb = zz[:, _H + _Z:_H + _Z + 1]

        ret = ret + (rb + sc @ wrs)

        rrow_s[...] = jnp.full((1, _NCHUNK * _CHUNK), -jnp.inf,
                               dtype=jnp.float32)
        rrow_s[:, 0:_CAND] = ret.reshape(1, _CAND)

        def rank_step(c, rank):
            base = c * _CHUNK
            rj = rrow_s[:, pl.ds(base, _CHUNK)]
            beats = (rj > ret) | ((rj == ret) & (jj + base < ii))
            return rank + jnp.sum(beats.astype(jnp.float32), axis=1,
                                  keepdims=True)

        rank = jax.lax.fori_loop(
            0, _NCHUNK, rank_step,
            jnp.zeros((_CAND, 1), dtype=jnp.float32))
        maskT = (rank < float(_TOP)).astype(jnp.float32).reshape(1, _CAND)

        if it + 1 < _ITERS:
            noise_cm = nc0_ref[0]
            ssum = jax.lax.dot(maskT, noise_cm, precision=hp)
            mean_n = ssum * (1.0 / _TOP)
            cent = noise_cm - mean_n
            msk2 = jax.lax.dot(maskT, cent * cent, precision=hp)
            std_n = jnp.sqrt(msk2 * (1.0 / _TOP))
            for h in range(_HORIZON):
                c0 = _ACTION * h
                mean_s[h] = mean_n[0, c0:c0 + _ACTION]
                std_s[h] = std_n[0, c0:c0 + _ACTION]
        else:
            s6 = jax.lax.dot(maskT, nh_ref[0, 0], precision=hp)
            out_ref[0, 0] = mean_s[0] + std_s[0] * (s6[0] * (1.0 / _TOP))


def kernel(belief, state, Wb, Ws, Wa, Wz, wrb, wrs):
    B, H = belief.shape
    Z = state.shape[1]

    key = jax.random.key(1234)
    nh = []
    for i in range(_ITERS):
        sub = jax.random.fold_in(key, i)
        n = jax.random.normal(sub, (_HORIZON, B, _CAND, _ACTION),
                              dtype=jnp.float32)
        nh.append(jnp.transpose(n, (1, 0, 2, 3)))
    nc0 = jnp.transpose(nh[0], (0, 2, 1, 3)).reshape(B, _CAND, _HA)

    WbZ = jnp.concatenate([Wb, Wz, wrb.reshape(H, 1)], axis=1)
    WsE = jnp.concatenate([Ws, wrs.reshape(Z, 1)], axis=1)

    full = lambda *shape: pl.BlockSpec(shape, lambda b: (0,) * len(shape))

    out = pl.pallas_call(
        _planner_body,
        grid=(B,),
        in_specs=[
            pl.BlockSpec((1, 1, H), lambda b: (b, 0, 0)),
            pl.BlockSpec((1, 1, Z), lambda b: (b, 0, 0)),
            full(H, H + Z + 1),
            full(Z, H + 1),
            full(_ACTION, H),
            full(Z, 1),
            pl.BlockSpec((1, _HORIZON, _CAND, _ACTION),
                         lambda b: (b, 0, 0, 0)),
            pl.BlockSpec((1, _HORIZON, _CAND, _ACTION),
                         lambda b: (b, 0, 0, 0)),
            pl.BlockSpec((1, _CAND, _HA), lambda b: (b, 0, 0)),
        ],
        out_specs=pl.BlockSpec((1, 1, _ACTION), lambda b: (b, 0, 0)),
        out_shape=jax.ShapeDtypeStruct((B, 1, _ACTION), jnp.float32),
        scratch_shapes=[
            pltpu.VMEM((_HORIZON, _ACTION), jnp.float32),
            pltpu.VMEM((_HORIZON, _ACTION), jnp.float32),
            pltpu.VMEM((1, _NCHUNK * _CHUNK), jnp.float32),
        ],
    )(belief.reshape(B, 1, H), state.reshape(B, 1, Z), WbZ, WsE, Wa,
      wrs.reshape(Z, 1), nh[0], nh[1], nc0)
    return out.reshape(B, _ACTION)


# speedup vs baseline: 6.5425x; 1.0388x over previous
import jax
import jax.numpy as jnp
from jax.experimental import pallas as pl
from jax.experimental.pallas import tpu as pltpu

_ACTION = 6
_HORIZON = 12
_ITERS = 2
_CAND = 1000
_TOP = 100
_HA = _HORIZON * _ACTION
_NCHUNK = 8
_CHUNK = 128
_H = 200
_Z = 30


def _planner_body(belief_ref, state_ref, WbZ_ref, WsE_ref, Wa_ref, wrs_ref,
                  nh0_ref, nh1_ref,
                  out_ref,
                  mean_s, std_s, rrow_s, ncm_s):
    WbZ = WbZ_ref[...]
    WsE = WsE_ref[...]
    Wa = Wa_ref[...]
    wrs = wrs_ref[...]

    hp = jax.lax.Precision.HIGHEST

    pre_shared = (belief_ref[0] @ WbZ[:, 0:_H]
                  + state_ref[0] @ WsE[:, 0:_H])

    ii = jax.lax.broadcasted_iota(jnp.int32, (_CAND, _CHUNK), 0)
    jj = jax.lax.broadcasted_iota(jnp.int32, (_CAND, _CHUNK), 1)

    for it in range(_ITERS):
        nh_ref = nh0_ref if it == 0 else nh1_ref

        if it == 0:
            a0 = nh_ref[0, 0]
            ncm_s[:, 0:_ACTION] = a0
        else:
            a0 = mean_s[0] + std_s[0] * nh_ref[0, 0]
        bc = jnp.tanh(pre_shared + a0 @ Wa)
        zz = bc @ WbZ
        sc = jnp.tanh(zz[:, _H:_H + _Z])
        bW = zz[:, 0:_H]
        rb = zz[:, _H + _Z:_H + _Z + 1]
        ret = None

        for h in range(1, _HORIZON):
            ss = sc @ WsE
            step_r = rb + ss[:, _H:_H + 1]
            ret = step_r if ret is None else ret + step_r
            if it == 0:
                a = nh_ref[h, 0]
                ncm_s[:, _ACTION * h:_ACTION * (h + 1)] = a
            else:
                a = mean_s[h] + std_s[h] * nh_ref[h, 0]
            pre = bW + ss[:, 0:_H] + a @ Wa
            bc = jnp.tanh(pre)
            zz = bc @ WbZ
            sc = jnp.tanh(zz[:, _H:_H + _Z])
            bW = zz[:, 0:_H]
            rb = zz[:, _H + _Z:_H + _Z + 1]

        ret = ret + (rb + sc @ wrs)

        rrow_s[...] = jnp.full((1, _NCHUNK * _CHUNK), -jnp.inf,
                               dtype=jnp.float32)
        rrow_s[:, 0:_CAND] = ret.reshape(1, _CAND)

        def rank_step(c, rank):
            base = c * _CHUNK
            rj = rrow_s[:, pl.ds(base, _CHUNK)]
            beats = (rj > ret) | ((rj == ret) & (jj + base < ii))
            return rank + jnp.sum(beats.astype(jnp.float32), axis=1,
                                  keepdims=True)

        rank = jax.lax.fori_loop(
            0, _NCHUNK, rank_step,
            jnp.zeros((_CAND, 1), dtype=jnp.float32))
        maskT = (rank < float(_TOP)).astype(jnp.float32).reshape(1, _CAND)

        if it + 1 < _ITERS:
            noise_cm = ncm_s[...]
            ssum = jax.lax.dot(maskT, noise_cm, precision=hp)
            mean_n = ssum * (1.0 / _TOP)
            cent = noise_cm - mean_n
            msk2 = jax.lax.dot(maskT, cent * cent, precision=hp)
            std_n = jnp.sqrt(msk2 * (1.0 / _TOP))
            for h in range(_HORIZON):
                c0 = _ACTION * h
                mean_s[h] = mean_n[0, c0:c0 + _ACTION]
                std_s[h] = std_n[0, c0:c0 + _ACTION]
        else:
            s6 = jax.lax.dot(maskT, nh_ref[0, 0], precision=hp)
            out_ref[0, 0] = mean_s[0] + std_s[0] * (s6[0] * (1.0 / _TOP))


def kernel(belief, state, Wb, Ws, Wa, Wz, wrb, wrs):
    B, H = belief.shape
    Z = state.shape[1]

    key = jax.random.key(1234)
    nh = []
    for i in range(_ITERS):
        sub = jax.random.fold_in(key, i)
        nh.append(jax.random.normal(sub, (_HORIZON, B, _CAND, _ACTION),
                                    dtype=jnp.float32))

    WbZ = jnp.concatenate([Wb, Wz, wrb.reshape(H, 1)], axis=1)
    WsE = jnp.concatenate([Ws, wrs.reshape(Z, 1)], axis=1)

    full = lambda *shape: pl.BlockSpec(shape, lambda b: (0,) * len(shape))

    out = pl.pallas_call(
        _planner_body,
        grid=(B,),
        in_specs=[
            pl.BlockSpec((1, 1, H), lambda b: (b, 0, 0)),
            pl.BlockSpec((1, 1, Z), lambda b: (b, 0, 0)),
            full(H, H + Z + 1),
            full(Z, H + 1),
            full(_ACTION, H),
            full(Z, 1),
            pl.BlockSpec((_HORIZON, 1, _CAND, _ACTION),
                         lambda b: (0, b, 0, 0)),
            pl.BlockSpec((_HORIZON, 1, _CAND, _ACTION),
                         lambda b: (0, b, 0, 0)),
        ],
        out_specs=pl.BlockSpec((1, 1, _ACTION), lambda b: (b, 0, 0)),
        out_shape=jax.ShapeDtypeStruct((B, 1, _ACTION), jnp.float32),
        scratch_shapes=[
            pltpu.VMEM((_HORIZON, _ACTION), jnp.float32),
            pltpu.VMEM((_HORIZON, _ACTION), jnp.float32),
            pltpu.VMEM((1, _NCHUNK * _CHUNK), jnp.float32),
            pltpu.VMEM((_CAND, _HA), jnp.float32),
        ],
    )(belief.reshape(B, 1, H), state.reshape(B, 1, Z), WbZ, WsE, Wa,
      wrs.reshape(Z, 1), nh[0], nh[1])
    return out.reshape(B, _ACTION)


# speedup vs baseline: 22.6914x; 3.4683x over previous
import jax
import jax.numpy as jnp
from jax.experimental import pallas as pl
from jax.experimental.pallas import tpu as pltpu

_ACTION = 6
_HORIZON = 12
_ITERS = 2
_CAND = 1000
_TOP = 100
_HA = _HORIZON * _ACTION
_NCHUNK = 8
_CHUNK = 128
_H = 200
_Z = 30
_B = 16

_KEYS = ((0x4B665424, 0x9617674F), (0xAB7D1D1B, 0x652FBEF2))


def _gen_noise(lconst, b_off, k0, k1):
    u32 = jnp.uint32
    ks0 = u32(k0)
    ks1 = u32(k1)
    ks2 = u32(k0 ^ k1 ^ 0x1BD11BDA)
    ri = jax.lax.broadcasted_iota(u32, (_CAND, _HA), 0)
    ctr = ri * u32(_ACTION) + lconst + b_off
    x0 = jnp.full((_CAND, _HA), ks0, dtype=u32)
    x1 = ctr + ks1
    rots = ((13, 15, 26, 6), (17, 29, 16, 24))
    inj = ((ks1, ks2), (ks2, ks0), (ks0, ks1), (ks1, ks2), (ks2, ks0))
    for g in range(5):
        for r in rots[g % 2]:
            x0 = x0 + x1
            x1 = ((x1 << u32(r)) | (x1 >> u32(32 - r))) ^ x0
        ia, ib = inj[g]
        x0 = x0 + ia
        x1 = x1 + ib + u32(g + 1)
    bits = x0 ^ x1
    fl = jax.lax.bitcast_convert_type((bits >> u32(9)) | u32(0x3F800000),
                                      jnp.float32)
    lo = jnp.float32(-0.9999999403953552)
    rng = jnp.float32(1.0) - lo
    u = jnp.maximum(lo, (fl - jnp.float32(1.0)) * rng + lo)
    return jnp.float32(1.4142135381698608) * jax.lax.erf_inv(u)


def _planner_body(belief_ref, state_ref, WbZ_ref, WsE_ref, Wa_ref, wrs_ref,
                  lconst_ref, out_ref,
                  mean_s, std_s, rrow_s, noise_s):
    WbZ = WbZ_ref[...]
    WsE = WsE_ref[...]
    Wa = Wa_ref[...]
    wrs = wrs_ref[...]
    lconst = lconst_ref[...]

    hp = jax.lax.Precision.HIGHEST
    b_off = (pl.program_id(0) * (_CAND * _ACTION)).astype(jnp.uint32)

    pre_shared = (belief_ref[0] @ WbZ[:, 0:_H]
                  + state_ref[0] @ WsE[:, 0:_H])

    ii = jax.lax.broadcasted_iota(jnp.int32, (_CAND, _CHUNK), 0)
    jj = jax.lax.broadcasted_iota(jnp.int32, (_CAND, _CHUNK), 1)

    for it in range(_ITERS):
        noise_s[...] = _gen_noise(lconst, b_off, *_KEYS[it])

        if it == 0:
            a0 = noise_s[:, 0:_ACTION]
        else:
            a0 = mean_s[0] + std_s[0] * noise_s[:, 0:_ACTION]
        bc = jnp.tanh(pre_shared + a0 @ Wa)
        zz = bc @ WbZ
        sc = jnp.tanh(zz[:, _H:_H + _Z])
        bW = zz[:, 0:_H]
        rb = zz[:, _H + _Z:_H + _Z + 1]
        ret = None

        for h in range(1, _HORIZON):
            ss = sc @ WsE
            step_r = rb + ss[:, _H:_H + 1]
            ret = step_r if ret is None else ret + step_r
            nh = noise_s[:, _ACTION * h:_ACTION * (h + 1)]
            if it == 0:
                a = nh
            else:
                a = mean_s[h] + std_s[h] * nh
            pre = bW + ss[:, 0:_H] + a @ Wa
            bc = jnp.tanh(pre)
            zz = bc @ WbZ
            sc = jnp.tanh(zz[:, _H:_H + _Z])
            bW = zz[:, 0:_H]
            rb = zz[:, _H + _Z:_H + _Z + 1]

        ret = ret + (rb + sc @ wrs)

        rrow_s[...] = jnp.full((1, _NCHUNK * _CHUNK), -jnp.inf,
                               dtype=jnp.float32)
        rrow_s[:, 0:_CAND] = ret.reshape(1, _CAND)

        def rank_step(c, rank):
            base = c * _CHUNK
            rj = rrow_s[:, pl.ds(base, _CHUNK)]
            beats = (rj > ret) | ((rj == ret) & (jj + base < ii))
            return rank + jnp.sum(beats.astype(jnp.float32), axis=1,
                                  keepdims=True)

        rank = jax.lax.fori_loop(
            0, _NCHUNK, rank_step,
            jnp.zeros((_CAND, 1), dtype=jnp.float32))
        maskT = (rank < float(_TOP)).astype(jnp.float32).reshape(1, _CAND)

        if it + 1 < _ITERS:
            noise_cm = noise_s[...]
            ssum = jax.lax.dot(maskT, noise_cm, precision=hp)
            mean_n = ssum * (1.0 / _TOP)
            cent = noise_cm - mean_n
            msk2 = jax.lax.dot(maskT, cent * cent, precision=hp)
            std_n = jnp.sqrt(msk2 * (1.0 / _TOP))
            for h in range(_HORIZON):
                c0 = _ACTION * h
                mean_s[h] = mean_n[0, c0:c0 + _ACTION]
                std_s[h] = std_n[0, c0:c0 + _ACTION]
        else:
            s6 = jax.lax.dot(maskT, noise_s[:, 0:_ACTION],
                             precision=hp)
            out_ref[0, 0] = mean_s[0] + std_s[0] * (s6[0] * (1.0 / _TOP))


def kernel(belief, state, Wb, Ws, Wa, Wz, wrb, wrs):
    B, H = belief.shape
    Z = state.shape[1]

    WbZ = jnp.concatenate([Wb, Wz, wrb.reshape(H, 1)], axis=1)
    WsE = jnp.concatenate([Ws, wrs.reshape(Z, 1)], axis=1)

    cols = jnp.arange(_HA, dtype=jnp.uint32)
    lconst = ((cols // _ACTION) * jnp.uint32(B * _CAND * _ACTION)
              + cols % _ACTION).reshape(1, _HA)

    full = lambda *shape: pl.BlockSpec(shape, lambda b: (0,) * len(shape))

    out = pl.pallas_call(
        _planner_body,
        grid=(B,),
        in_specs=[
            pl.BlockSpec((1, 1, H), lambda b: (b, 0, 0)),
            pl.BlockSpec((1, 1, Z), lambda b: (b, 0, 0)),
            full(H, H + Z + 1),
            full(Z, H + 1),
            full(_ACTION, H),
            full(Z, 1),
            full(1, _HA),
        ],
        out_specs=pl.BlockSpec((1, 1, _ACTION), lambda b: (b, 0, 0)),
        out_shape=jax.ShapeDtypeStruct((B, 1, _ACTION), jnp.float32),
        scratch_shapes=[
            pltpu.VMEM((_HORIZON, _ACTION), jnp.float32),
            pltpu.VMEM((_HORIZON, _ACTION), jnp.float32),
            pltpu.VMEM((1, _NCHUNK * _CHUNK), jnp.float32),
            pltpu.VMEM((_CAND, _HA), jnp.float32),
        ],
    )(belief.reshape(B, 1, H), state.reshape(B, 1, Z), WbZ, WsE, Wa,
      wrs.reshape(Z, 1), lconst)
    return out.reshape(B, _ACTION)


# speedup vs baseline: 26.5364x; 1.1694x over previous
import jax
import jax.numpy as jnp
from jax.experimental import pallas as pl
from jax.experimental.pallas import tpu as pltpu

_ACTION = 6
_HORIZON = 12
_ITERS = 2
_CAND = 1000
_TOP = 100
_HA = _HORIZON * _ACTION
_NCHUNK = 8
_CHUNK = 128
_H = 200
_Z = 30
_B = 16

_KEYS = ((0x4B665424, 0x9617674F), (0xAB7D1D1B, 0x652FBEF2))


def _gen_noise(lconst, b_off, k0, k1):
    u32 = jnp.uint32
    ks0 = u32(k0)
    ks1 = u32(k1)
    ks2 = u32(k0 ^ k1 ^ 0x1BD11BDA)
    ri = jax.lax.broadcasted_iota(u32, (_CAND, _HA), 0)
    ctr = ri * u32(_ACTION) + lconst + b_off
    x0 = jnp.full((_CAND, _HA), ks0, dtype=u32)
    x1 = ctr + ks1
    rots = ((13, 15, 26, 6), (17, 29, 16, 24))
    inj = ((ks1, ks2), (ks2, ks0), (ks0, ks1), (ks1, ks2), (ks2, ks0))
    for g in range(5):
        for r in rots[g % 2]:
            x0 = x0 + x1
            x1 = ((x1 << u32(r)) | (x1 >> u32(32 - r))) ^ x0
        ia, ib = inj[g]
        x0 = x0 + ia
        x1 = x1 + ib + u32(g + 1)
    bits = x0 ^ x1
    fl = jax.lax.bitcast_convert_type((bits >> u32(9)) | u32(0x3F800000),
                                      jnp.float32)
    lo = jnp.float32(-0.9999999403953552)
    rng = jnp.float32(1.0) - lo
    u = jnp.maximum(lo, (fl - jnp.float32(1.0)) * rng + lo)
    return jnp.float32(1.4142135381698608) * jax.lax.erf_inv(u)


def _planner_body(belief_ref, state_ref, WbZ_ref, WsE_ref, Wa_ref, wrs_ref,
                  lconst_ref, out_ref,
                  mean_s, std_s, rrow_s, noise0_s, noise1_s):
    WbZ = WbZ_ref[...]
    WsE = WsE_ref[...]
    Wa = Wa_ref[...]
    wrs = wrs_ref[...]
    lconst = lconst_ref[...]

    hp = jax.lax.Precision.HIGHEST
    b_off = (pl.program_id(0) * (_CAND * _ACTION)).astype(jnp.uint32)

    pre_shared = (belief_ref[0] @ WbZ[:, 0:_H]
                  + state_ref[0] @ WsE[:, 0:_H])

    ii = jax.lax.broadcasted_iota(jnp.int32, (_CAND, _CHUNK), 0)
    jj = jax.lax.broadcasted_iota(jnp.int32, (_CAND, _CHUNK), 1)

    noise0_s[...] = _gen_noise(lconst, b_off, *_KEYS[0])
    noise1_s[...] = _gen_noise(lconst, b_off, *_KEYS[1])

    for it in range(_ITERS):
        noise_s = noise0_s if it == 0 else noise1_s

        if it == 0:
            a0 = noise_s[:, 0:_ACTION]
        else:
            a0 = mean_s[0] + std_s[0] * noise_s[:, 0:_ACTION]
        bc = jnp.tanh(pre_shared + a0 @ Wa)
        zz = bc @ WbZ
        sc = jnp.tanh(zz[:, _H:_H + _Z])
        bW = zz[:, 0:_H]
        rb = zz[:, _H + _Z:_H + _Z + 1]
        ret = None

        for h in range(1, _HORIZON):
            ss = sc @ WsE
            step_r = rb + ss[:, _H:_H + 1]
            ret = step_r if ret is None else ret + step_r
            nh = noise_s[:, _ACTION * h:_ACTION * (h + 1)]
            if it == 0:
                a = nh
            else:
                a = mean_s[h] + std_s[h] * nh
            pre = bW + ss[:, 0:_H] + a @ Wa
            bc = jnp.tanh(pre)
            zz = bc @ WbZ
            sc = jnp.tanh(zz[:, _H:_H + _Z])
            bW = zz[:, 0:_H]
            rb = zz[:, _H + _Z:_H + _Z + 1]

        ret = ret + (rb + sc @ wrs)

        rrow_s[...] = jnp.full((1, _NCHUNK * _CHUNK), -jnp.inf,
                               dtype=jnp.float32)
        rrow_s[:, 0:_CAND] = ret.reshape(1, _CAND)

        rank = jnp.zeros((_CAND, 1), dtype=jnp.float32)
        for c in range(_NCHUNK):
            base = c * _CHUNK
            rj = rrow_s[:, base:base + _CHUNK]
            beats = (rj > ret) | ((rj == ret) & (jj + base < ii))
            rank = rank + jnp.sum(beats.astype(jnp.float32), axis=1,
                                  keepdims=True)
        maskT = (rank < float(_TOP)).astype(jnp.float32).reshape(1, _CAND)

        if it + 1 < _ITERS:
            noise_cm = noise_s[...]
            ssum = jax.lax.dot(maskT, noise_cm, precision=hp)
            mean_n = ssum * (1.0 / _TOP)
            cent = noise_cm - mean_n
            msk2 = jax.lax.dot(maskT, cent * cent, precision=hp)
            std_n = jnp.sqrt(msk2 * (1.0 / _TOP))
            for h in range(_HORIZON):
                c0 = _ACTION * h
                mean_s[h] = mean_n[0, c0:c0 + _ACTION]
                std_s[h] = std_n[0, c0:c0 + _ACTION]
        else:
            s6 = jax.lax.dot(maskT, noise_s[:, 0:_ACTION],
                             precision=hp)
            out_ref[0, 0] = mean_s[0] + std_s[0] * (s6[0] * (1.0 / _TOP))


def kernel(belief, state, Wb, Ws, Wa, Wz, wrb, wrs):
    B, H = belief.shape
    Z = state.shape[1]

    WbZ = jnp.concatenate([Wb, Wz, wrb.reshape(H, 1)], axis=1)
    WsE = jnp.concatenate([Ws, wrs.reshape(Z, 1)], axis=1)

    cols = jnp.arange(_HA, dtype=jnp.uint32)
    lconst = ((cols // _ACTION) * jnp.uint32(B * _CAND * _ACTION)
              + cols % _ACTION).reshape(1, _HA)

    full = lambda *shape: pl.BlockSpec(shape, lambda b: (0,) * len(shape))

    out = pl.pallas_call(
        _planner_body,
        grid=(B,),
        in_specs=[
            pl.BlockSpec((1, 1, H), lambda b: (b, 0, 0)),
            pl.BlockSpec((1, 1, Z), lambda b: (b, 0, 0)),
            full(H, H + Z + 1),
            full(Z, H + 1),
            full(_ACTION, H),
            full(Z, 1),
            full(1, _HA),
        ],
        out_specs=pl.BlockSpec((1, 1, _ACTION), lambda b: (b, 0, 0)),
        out_shape=jax.ShapeDtypeStruct((B, 1, _ACTION), jnp.float32),
        scratch_shapes=[
            pltpu.VMEM((_HORIZON, _ACTION), jnp.float32),
            pltpu.VMEM((_HORIZON, _ACTION), jnp.float32),
            pltpu.VMEM((1, _NCHUNK * _CHUNK), jnp.float32),
            pltpu.VMEM((_CAND, _HA), jnp.float32),
            pltpu.VMEM((_CAND, _HA), jnp.float32),
        ],
    )(belief.reshape(B, 1, H), state.reshape(B, 1, Z), WbZ, WsE, Wa,
      wrs.reshape(Z, 1), lconst)
    return out.reshape(B, _ACTION)
